# baseline (speedup 1.0000x reference)
import jax
import jax.numpy as jnp
from jax.experimental import pallas as pl
from jax.experimental.pallas import tpu as pltpu

N = 4096
BLOCK_ROWS = 256


def _softmax_rows(x_ref, o_ref, t_ref):
    t_ref[...] = x_ref[...].reshape(BLOCK_ROWS, N)
    x = t_ref[...]
    m = jnp.max(x, axis=1, keepdims=True)
    e = jnp.exp(x - m)
    s = jnp.sum(e, axis=1, keepdims=True)
    o_ref[...] = e / s


def kernel(free_params, free_row_idx, free_col_idx):
    del free_row_idx, free_col_idx
    return pl.pallas_call(
        _softmax_rows,
        grid=(N // BLOCK_ROWS,),
        in_specs=[pl.BlockSpec((BLOCK_ROWS * N,), lambda i: (i,))],
        out_specs=pl.BlockSpec((BLOCK_ROWS, N), lambda i: (i, 0)),
        out_shape=jax.ShapeDtypeStruct((N, N), jnp.float32),
        scratch_shapes=[pltpu.VMEM((BLOCK_ROWS, N), jnp.float32)],
    )(free_params)


# speedup vs baseline: 1.0354x; 1.0354x over previous
import jax
import jax.numpy as jnp
from jax.experimental import pallas as pl

N = 4096
BLOCK_ROWS = 512


def _softmax_rows(x_ref, o_ref):
    o_ref[...] = x_ref[...].reshape(BLOCK_ROWS, N)
    x = o_ref[...]
    m = jnp.max(x, axis=1, keepdims=True)
    e = jnp.exp(x - m)
    s = jnp.sum(e, axis=1, keepdims=True)
    o_ref[...] = e / s


def kernel(free_params, free_row_idx, free_col_idx):
    del free_row_idx, free_col_idx
    return pl.pallas_call(
        _softmax_rows,
        grid=(N // BLOCK_ROWS,),
        in_specs=[pl.BlockSpec((BLOCK_ROWS * N,), lambda i: (i,))],
        out_specs=pl.BlockSpec((BLOCK_ROWS, N), lambda i: (i, 0)),
        out_shape=jax.ShapeDtypeStruct((N, N), jnp.float32),
    )(free_params)
